# baseline (speedup 1.0000x reference)
import functools

import jax
import jax.numpy as jnp
from jax import lax
from jax.experimental import pallas as pl
from jax.experimental.pallas import tpu as pltpu
from jax.experimental.pallas import tpu_sc as plsc

NREAL = 10000
NPAD = 10240
E = 320000
EMB = 128
G = 128
HIP = lax.Precision.HIGHEST
F32 = jnp.float32

NC, NS = 2, 16
CHUNK = 128
NCH = 80
EPW = CHUNK * NCH
EPC = EPW * NS
EPAD = EPC * NC
STRIPE = NPAD // NS

_KEY_N1 = -1065353217
_KEY_P1 = 1065353216


def _rows(n, w=1):
    return lax.broadcasted_iota(jnp.int32, (n, w), 0)


def _shift_down(a, d, fill):
    r = pltpu.roll(a, d, 0)
    return jnp.where(_rows(a.shape[0], a.shape[1]) < d, fill, r)


def _shift_up(a, d, fill):
    r = pltpu.roll(a, a.shape[0] - d, 0)
    return jnp.where(_rows(a.shape[0], a.shape[1]) >= a.shape[0] - d, fill, r)


def _pre_body(x_ref, dega_ref, degb_ref, m_ref, w_ref, dinv_ref, u_ref, acc_ref):
    x = x_ref[...]
    deg = (dega_ref[:, 0:1] + degb_ref[:, 0:1]) * m_ref[...]
    dinv = jnp.where(deg > 0, lax.rsqrt(jnp.maximum(deg, 1e-30)), 0.0)
    dinv_ref[...] = dinv
    u_ref[...] = x * dinv
    acc_ref[...] = jnp.dot(x, w_ref[...], preferred_element_type=F32)


def _pre_call(x, dega, degb, m, w, interpret=False):
    return pl.pallas_call(
        _pre_body,
        out_shape=[jax.ShapeDtypeStruct((NPAD, 1), F32),
                   jax.ShapeDtypeStruct((NPAD, EMB), F32),
                   jax.ShapeDtypeStruct((NPAD, EMB), F32)],
        interpret=interpret,
    )(x, dega, degb, m, w)


def _mid_body(a0_ref, a1_ref, dinv_ref, acc_ref, w_ref, accn_ref, u_ref):
    dinv = dinv_ref[...]
    tx1 = -dinv * (a0_ref[...] + a1_ref[...])
    accn_ref[...] = acc_ref[...] + jnp.dot(tx1, w_ref[...], preferred_element_type=F32)
    u_ref[...] = dinv * tx1


def _mid_call(a0, a1, dinv, acc, w, interpret=False):
    return pl.pallas_call(
        _mid_body,
        out_shape=[jax.ShapeDtypeStruct((NPAD, EMB), F32),
                   jax.ShapeDtypeStruct((NPAD, EMB), F32)],
        interpret=interpret,
    )(a0, a1, dinv, acc, w)


def _tail_common(a0, a1, dinv, acc, x0, w2, b, lw, lb, bg, bb, m):
    tx2 = -2.0 * dinv * (a0 + a1) - x0
    h = acc + jnp.dot(tx2, w2, preferred_element_type=F32) + b
    y = jnp.dot(h, lw, preferred_element_type=F32) + lb
    y = jnp.maximum(y, 0.0)
    cnt = jnp.maximum(jnp.sum(m), 1.0)
    mean = jnp.sum(y * m, axis=0, keepdims=True) / cnt
    var = jnp.sum(((y - mean) ** 2) * m, axis=0, keepdims=True) / cnt
    return bg * (y - mean) * lax.rsqrt(var + 1e-5) + bb


def _tail0_body(a0_ref, a1_ref, dinv_ref, acc_ref, x0_ref, w2_ref, b_ref,
                lw_ref, lb_ref, bg_ref, bb_ref, m_ref, wn_ref,
                x1_ref, u_ref, accn_ref):
    dinv = dinv_ref[...]
    xbn = _tail_common(a0_ref[...], a1_ref[...], dinv, acc_ref[...], x0_ref[...],
                       w2_ref[...], b_ref[...], lw_ref[...], lb_ref[...],
                       bg_ref[...], bb_ref[...], m_ref[...])
    x1_ref[...] = xbn
    u_ref[...] = dinv * xbn
    accn_ref[...] = jnp.dot(xbn, wn_ref[...], preferred_element_type=F32)


def _tail0_call(a0, a1, dinv, acc, x0, w2, b, lw, lb, bg, bb, m, wn, interpret=False):
    return pl.pallas_call(
        _tail0_body,
        out_shape=[jax.ShapeDtypeStruct((NPAD, EMB), F32),
                   jax.ShapeDtypeStruct((NPAD, EMB), F32),
                   jax.ShapeDtypeStruct((NPAD, EMB), F32)],
        interpret=interpret,
    )(a0, a1, dinv, acc, x0, w2, b, lw, lb, bg, bb, m, wn)


def _gather_graph_i32(oh, v):
    hi16 = lax.shift_right_arithmetic(v, 16)
    lo16 = v & jnp.int32(0xFFFF)
    hl = jnp.concatenate([hi16, lo16], axis=0).astype(F32)
    gn = jnp.dot(oh, hl.T, precision=HIP,
                 preferred_element_type=F32).astype(jnp.int32)
    return (gn[:, 0:1] << 16) | gn[:, 1:2]


def _colsum(vec, oh):
    return lax.dot_general(vec, oh, (((0,), (0,)), ((), ())), precision=HIP,
                           preferred_element_type=F32)


def _score_body(xbn_ref, pw_ref, m_ref, score_ref, key_ref):
    pw = pw_ref[...]
    wnorm = jnp.sqrt(jnp.sum(pw * pw))
    score = jnp.tanh(jnp.sum(xbn_ref[...] * pw, axis=1, keepdims=True) / wnorm)
    kb = lax.bitcast_convert_type(score, jnp.int32)
    key = jnp.where(kb < 0, kb ^ jnp.int32(0x7FFFFFFF), kb)
    score_ref[...] = score
    key_ref[...] = jnp.where(m_ref[...] > 0, key, jnp.int32(-2147483648))


def _score_call(xbn, pw, m, interpret=False):
    return pl.pallas_call(
        _score_body,
        out_shape=[jax.ShapeDtypeStruct((NPAD, 1), F32),
                   jax.ShapeDtypeStruct((NPAD, 1), jnp.int32)],
        interpret=interpret,
    )(xbn, pw, m)


def _thresh_body(key_ref, m_ref, batch_ref, t_ref, kk_ref):
    key = key_ref[...]
    oh = (batch_ref[...] == lax.broadcasted_iota(jnp.int32, (NPAD, G), 1)).astype(F32)
    size = _colsum(m_ref[...], oh)
    kk = jnp.floor((size + 1.0) * 0.5)

    def body(_, lh):
        lo, hi = lh
        mid = lo + (hi - lo) // 2
        midn = _gather_graph_i32(oh, mid)
        ind = jnp.where(key >= midn, 1.0, 0.0)
        cnt = _colsum(ind, oh)
        cond = cnt >= kk
        return jnp.where(cond, mid, lo), jnp.where(cond, hi, mid)

    lo0 = jnp.full((1, G), _KEY_N1, jnp.int32)
    hi0 = jnp.full((1, G), _KEY_P1 + 1, jnp.int32)
    t, _ = lax.fori_loop(0, 32, body, (lo0, hi0))
    t_ref[...] = t
    kk_ref[...] = kk


def _thresh_call(key, m, batch, interpret=False):
    return pl.pallas_call(
        _thresh_body,
        out_shape=[jax.ShapeDtypeStruct((1, G), jnp.int32),
                   jax.ShapeDtypeStruct((1, G), F32)],
        interpret=interpret,
    )(key, m, batch)


def _tiecnt_body(key_ref, batch_ref, t_ref, kk_ref, code_ref, pn_ref):
    key = key_ref[...]
    oh = (batch_ref[...] == lax.broadcasted_iota(jnp.int32, (NPAD, G), 1)).astype(F32)
    tn = _gather_graph_i32(oh, t_ref[...])
    gtb = (key > tn).astype(F32)
    tieb = (key == tn).astype(F32)
    cnt_gt = _colsum(gtb, oh)
    cnt_tie = _colsum(tieb, oh)
    need = kk_ref[...] - cnt_gt
    gg = lax.broadcasted_iota(jnp.int32, (G, G), 0)
    su = (gg < lax.broadcasted_iota(jnp.int32, (G, G), 1)).astype(F32)
    off = jnp.dot(cnt_tie, su, precision=HIP, preferred_element_type=F32)
    code_ref[...] = 2.0 * gtb + tieb
    pn_ref[...] = jnp.dot(oh, (off + need).T, precision=HIP,
                          preferred_element_type=F32)


def _tiecnt_call(key, batch, t, kk, interpret=False):
    return pl.pallas_call(
        _tiecnt_body,
        out_shape=[jax.ShapeDtypeStruct((NPAD, 1), F32),
                   jax.ShapeDtypeStruct((NPAD, 1), F32)],
        interpret=interpret,
    )(key, batch, t, kk)


def _keep_body(code_ref, pn_ref, keep_ref):
    code = code_ref[...]
    tie_i = jnp.where((code == 1.0) | (code == 3.0), 1.0, 0.0)

    def cs_body(i, cs):
        d = jnp.int32(1) << i
        return cs + jnp.where(_rows(NPAD, 1) < d, 0.0, pltpu.roll(cs, d, 0))

    cs = lax.fori_loop(0, 14, cs_body, tie_i)
    keep = (code >= 2.0) | ((tie_i > 0) & (cs - tie_i < pn_ref[...]))
    keep_ref[...] = keep.astype(F32)


def _keep_call(code, pn, interpret=False):
    return pl.pallas_call(
        _keep_body,
        out_shape=jax.ShapeDtypeStruct((NPAD, 1), F32),
        interpret=interpret,
    )(code, pn)


def _gap_body(xbn_ref, score_ref, keep_ref, batch_ref, xp_ref, gapcnt_ref):
    keepf = keep_ref[...]
    xp = xbn_ref[...] * score_ref[...] * keepf
    oh = (batch_ref[...] == lax.broadcasted_iota(jnp.int32, (NPAD, G), 1)).astype(F32)
    cnt_a = lax.dot_general(oh, keepf, (((0,), (0,)), ((), ())), precision=HIP,
                            preferred_element_type=F32)
    ssum = lax.dot_general(oh, xp, (((0,), (0,)), ((), ())), precision=HIP,
                           preferred_element_type=F32)
    gap = ssum / jnp.maximum(cnt_a, 1.0)
    xp_ref[...] = xp
    gapcnt_ref[...] = jnp.concatenate([gap, cnt_a], axis=1)


def _gap_call(xbn, score, keep, batch, interpret=False):
    return pl.pallas_call(
        _gap_body,
        out_shape=[jax.ShapeDtypeStruct((NPAD, EMB), F32),
                   jax.ShapeDtypeStruct((G, EMB + 1), F32)],
        interpret=interpret,
    )(xbn, score, keep, batch)


def _gmp_body(xp_ref, keep_ref, batch_ref, gapcnt_ref, repacc_ref, rep_ref):
    batch = batch_ref[...]
    xp = xp_ref[...]

    def scan_body(i, f):
        d = jnp.int32(1) << i
        bs = jnp.where(_rows(NPAD, 1) < d, jnp.int32(-1), pltpu.roll(batch, d, 0))
        fs = jnp.where(_rows(NPAD, EMB) < d, -1e30, pltpu.roll(f, d, 0))
        return jnp.where(bs == batch, jnp.maximum(f, fs), f)

    f = lax.fori_loop(0, 14, scan_body,
                      jnp.where(keep_ref[...] > 0, xp, -1e30))
    oh = (batch == lax.broadcasted_iota(jnp.int32, (NPAD, G), 1)).astype(F32)
    lastf = (batch != _shift_up(batch, 1, jnp.int32(-2))).astype(F32)
    gmp = lax.dot_general(oh * lastf, f, (((0,), (0,)), ((), ())), precision=HIP,
                          preferred_element_type=F32)
    gapcnt = gapcnt_ref[...]
    cnt_a = gapcnt[:, EMB:EMB + 1]
    gmp = jnp.where(cnt_a > 0.5, gmp, 0.0)
    rep = jnp.concatenate([gmp, gapcnt[:, :EMB]], axis=1)
    rep_ref[...] = repacc_ref[...] + rep


def _gmp_call(xp, keep, batch, gapcnt, repacc, interpret=False):
    return pl.pallas_call(
        _gmp_body,
        out_shape=jax.ShapeDtypeStruct((G, 2 * EMB), F32),
        interpret=interpret,
    )(xp, keep, batch, gapcnt, repacc)


def _pool_call(xbn, m, batch, pw, repacc, interpret=False):
    score, key = _score_call(xbn, pw, m, interpret=interpret)
    t, kk = _thresh_call(key, m, batch, interpret=interpret)
    code, pn = _tiecnt_call(key, batch, t, kk, interpret=interpret)
    keep = _keep_call(code, pn, interpret=interpret)
    xp, gapcnt = _gap_call(xbn, score, keep, batch, interpret=interpret)
    rep = _gmp_call(xp, keep, batch, gapcnt, repacc, interpret=interpret)
    return xp, keep, rep


def _tailbn_body(a0_ref, a1_ref, dinv_ref, acc_ref, x0_ref, w2_ref, b_ref,
                 lw_ref, lb_ref, bg_ref, bb_ref, m_ref, xbn_ref):
    xbn_ref[...] = _tail_common(a0_ref[...], a1_ref[...], dinv_ref[...],
                                acc_ref[...], x0_ref[...], w2_ref[...],
                                b_ref[...], lw_ref[...], lb_ref[...],
                                bg_ref[...], bb_ref[...], m_ref[...])


def _tailbn_call(a0, a1, dinv, acc, x0, w2, b, lw, lb, bg, bb, m, interpret=False):
    return pl.pallas_call(
        _tailbn_body,
        out_shape=jax.ShapeDtypeStruct((NPAD, EMB), F32),
        interpret=interpret,
    )(a0, a1, dinv, acc, x0, w2, b, lw, lb, bg, bb, m)


def _zero_vmem_rows(buf, nrows, width):
    z = jnp.zeros((16,), F32)

    def row(i, _):
        for j in range(width // 16):
            buf[i, pl.ds(j * 16, 16)] = z
        return 0

    lax.fori_loop(0, nrows, row, 0)


@functools.lru_cache(maxsize=None)
def _sc_kernels():
    mesh = plsc.VectorSubcoreMesh(core_axis_name="c", subcore_axis_name="s")

    @functools.partial(
        pl.kernel, mesh=mesh,
        out_type=[jax.ShapeDtypeStruct((NPAD, EMB), F32),
                  jax.ShapeDtypeStruct((NPAD, EMB), F32)],
        scratch_types=[pltpu.VMEM((CHUNK,), jnp.int32),
                       pltpu.VMEM((CHUNK,), jnp.int32),
                       pltpu.VMEM((NCH, CHUNK), jnp.int32),
                       pltpu.VMEM((CHUNK, EMB), F32),
                       pltpu.VMEM((CHUNK, EMB), F32),
                       pltpu.VMEM_SHARED((NPAD, EMB), F32),
                       pltpu.SemaphoreType.DMA,
                       pltpu.SemaphoreType.DMA,
                       pltpu.SemaphoreType.DMA,
                       pltpu.SemaphoreType.DMA],
    )
    def _lap_kernel(u_hbm, src1_hbm, dst2_hbm, o0_hbm, o1_hbm,
                    siA, siB, di2, rows0, rows1, acc, semA, semB, semSA, semSB):
        cid = lax.axis_index("c")
        sid = lax.axis_index("s")
        _zero_vmem_rows(rows0, CHUNK, EMB)
        for j in range(STRIPE // CHUNK):
            pltpu.sync_copy(rows0, acc.at[pl.ds(sid * STRIPE + j * CHUNK, CHUNK)])
        rem = STRIPE - (STRIPE // CHUNK) * CHUNK
        if rem:
            pltpu.sync_copy(rows0.at[pl.ds(0, rem)],
                            acc.at[pl.ds(sid * STRIPE + STRIPE - rem, rem)])
        rb = cid * (EPC // CHUNK) + sid * NCH
        pltpu.sync_copy(dst2_hbm.at[pl.ds(rb, NCH)], di2)
        base = cid * EPC + sid * EPW
        plsc.subcore_barrier()

        pltpu.sync_copy(src1_hbm.at[pl.ds(base, CHUNK)], siA)
        pltpu.async_copy(u_hbm.at[siA], rows0, semA)

        @pl.when(1 < NCH)
        def _():
            pltpu.async_copy(src1_hbm.at[pl.ds(base + CHUNK, CHUNK)], siB, semSB)

        def step(j, _):
            i0 = 2 * j
            i1 = 2 * j + 1
            i2 = 2 * j + 2
            i3 = 2 * j + 3

            @pl.when(i1 < NCH)
            def _():
                pltpu.make_async_copy(src1_hbm.at[pl.ds(base + i1 * CHUNK, CHUNK)],
                                      siB, semSB).wait()
                pltpu.async_copy(u_hbm.at[siB], rows1, semB)

            pltpu.make_async_copy(u_hbm.at[siA], rows0, semA).wait()
            pltpu.sync_copy(rows0, acc.at[di2.at[i0]], add=True)

            @pl.when(i2 < NCH)
            def _():
                pltpu.async_copy(src1_hbm.at[pl.ds(base + i2 * CHUNK, CHUNK)],
                                 siA, semSA)
                pltpu.make_async_copy(src1_hbm.at[pl.ds(base + i2 * CHUNK, CHUNK)],
                                      siA, semSA).wait()
                pltpu.async_copy(u_hbm.at[siA], rows0, semA)

            @pl.when(i1 < NCH)
            def _():
                pltpu.make_async_copy(u_hbm.at[siB], rows1, semB).wait()
                pltpu.sync_copy(rows1, acc.at[di2.at[i1]], add=True)

            @pl.when(i3 < NCH)
            def _():
                pltpu.async_copy(src1_hbm.at[pl.ds(base + i3 * CHUNK, CHUNK)],
                                 siB, semSB)

            return 0

        lax.fori_loop(0, (NCH + 1) // 2, step, 0)
        plsc.subcore_barrier()

        @pl.when(cid == 0)
        def _():
            pltpu.sync_copy(acc.at[pl.ds(sid * STRIPE, STRIPE)],
                            o0_hbm.at[pl.ds(sid * STRIPE, STRIPE)])

        @pl.when(cid == 1)
        def _():
            pltpu.sync_copy(acc.at[pl.ds(sid * STRIPE, STRIPE)],
                            o1_hbm.at[pl.ds(sid * STRIPE, STRIPE)])

    return (_lap_kernel,)


def _lap_call(u, edges):
    src1, dst1, src2d, dst2d = edges
    return _sc_kernels()[0](u, src1, dst2d)


def _mdeg_call(m128, edges):
    src1, dst1, src2d, dst2d = edges
    return _sc_kernels()[0](m128, dst1, src2d)


def _forward(x, edge_index, batch_index, cheb_W, cheb_b, lin_W, lin_b,
             bn_g, bn_b, pool_w, lap_fn, mdeg_fn, interpret=False):
    xp = jnp.zeros((NPAD, EMB), F32).at[:NREAL].set(x)
    batch = jnp.full((NPAD, 1), G - 1, jnp.int32).at[:NREAL, 0].set(batch_index)
    m = jnp.zeros((NPAD, 1), F32).at[:NREAL].set(1.0)
    m128 = jnp.zeros((NPAD, EMB), F32).at[:NREAL].set(1.0)
    pad = jnp.full((EPAD - E,), NREAL, jnp.int32)
    src1 = jnp.concatenate([edge_index[0], pad])
    dst1 = jnp.concatenate([edge_index[1], pad])
    edges = (src1, dst1,
             src1.reshape(EPAD // CHUNK, CHUNK),
             dst1.reshape(EPAD // CHUNK, CHUNK))
    b2 = lambda v: v.reshape(1, EMB)

    def conv_to_tail(xin, dega, degb, m, w3):
        dinv, u, acc = _pre_call(xin, dega, degb, m, w3[0], interpret=interpret)
        a0, a1 = lap_fn(u, edges)
        acc, u = _mid_call(a0, a1, dinv, acc, w3[1], interpret=interpret)
        a0, a1 = lap_fn(u, edges)
        return dinv, acc, a0, a1

    rep = jnp.zeros((G, 2 * EMB), F32)
    dega, degb = mdeg_fn(m128, edges)
    dinv, acc, a0, a1 = conv_to_tail(xp, dega, degb, m, cheb_W[0])
    xp, u, acc = _tail0_call(a0, a1, dinv, acc, xp, cheb_W[0][2], b2(cheb_b[0]),
                             lin_W[0], b2(lin_b[0]), b2(bn_g[0]), b2(bn_b[0]),
                             m, cheb_W[1][0], interpret=interpret)
    a0, a1 = lap_fn(u, edges)
    acc, u = _mid_call(a0, a1, dinv, acc, cheb_W[1][1], interpret=interpret)
    a0, a1 = lap_fn(u, edges)
    xbn = _tailbn_call(a0, a1, dinv, acc, xp, cheb_W[1][2], b2(cheb_b[1]),
                       lin_W[1], b2(lin_b[1]), b2(bn_g[1]), b2(bn_b[1]), m,
                       interpret=interpret)
    xp, m, rep = _pool_call(xbn, m, batch, b2(pool_w[0]), rep,
                            interpret=interpret)
    m128 = jnp.broadcast_to(m, (NPAD, EMB))
    for i in (1, 2):
        dega, degb = mdeg_fn(m128, edges)
        dinv, acc, a0, a1 = conv_to_tail(xp, dega, degb, m, cheb_W[i + 1])
        xbn = _tailbn_call(a0, a1, dinv, acc, xp, cheb_W[i + 1][2],
                           b2(cheb_b[i + 1]), lin_W[i + 1], b2(lin_b[i + 1]),
                           b2(bn_g[i + 1]), b2(bn_b[i + 1]), m,
                           interpret=interpret)
        xp, m, rep = _pool_call(xbn, m, batch, b2(pool_w[i]), rep,
                                interpret=interpret)
        m128 = jnp.broadcast_to(m, (NPAD, EMB))
    return rep


def kernel(x, edge_attr, edge_index, batch_index, cheb_W, cheb_b, lin_W, lin_b,
           bn_g, bn_b, pool_w):
    return _forward(x, edge_index, batch_index, cheb_W, cheb_b, lin_W, lin_b,
                    bn_g, bn_b, pool_w, _lap_call, _mdeg_call)


# speedup vs baseline: 1.3229x; 1.3229x over previous
import functools

import jax
import jax.numpy as jnp
from jax import lax
from jax.experimental import pallas as pl
from jax.experimental.pallas import tpu as pltpu
from jax.experimental.pallas import tpu_sc as plsc

NREAL = 10000
NPAD = 10240
E = 320000
EMB = 128
G = 128
HIP = lax.Precision.HIGHEST
F32 = jnp.float32

NC, NS = 2, 16
CHUNK = 80
NCH = 126
EPW = CHUNK * NCH
EPC = EPW * NS
EPAD = EPC * NC
STRIPE = NPAD // NS

_KEY_N1 = -1065353217
_KEY_P1 = 1065353216


def _rows(n, w=1):
    return lax.broadcasted_iota(jnp.int32, (n, w), 0)


def _shift_down(a, d, fill):
    r = pltpu.roll(a, d, 0)
    return jnp.where(_rows(a.shape[0], a.shape[1]) < d, fill, r)


def _shift_up(a, d, fill):
    r = pltpu.roll(a, a.shape[0] - d, 0)
    return jnp.where(_rows(a.shape[0], a.shape[1]) >= a.shape[0] - d, fill, r)


def _pre_body(x_ref, dega_ref, degb_ref, m_ref, w_ref, dinv_ref, u_ref, acc_ref):
    x = x_ref[...]
    deg = (dega_ref[:, 0:1] + degb_ref[:, 0:1]) * m_ref[...]
    dinv = jnp.where(deg > 0, lax.rsqrt(jnp.maximum(deg, 1e-30)), 0.0)
    dinv_ref[...] = dinv
    u_ref[...] = x * dinv
    acc_ref[...] = jnp.dot(x, w_ref[...], preferred_element_type=F32)


def _pre_call(x, dega, degb, m, w, interpret=False):
    return pl.pallas_call(
        _pre_body,
        out_shape=[jax.ShapeDtypeStruct((NPAD, 1), F32),
                   jax.ShapeDtypeStruct((NPAD, EMB), F32),
                   jax.ShapeDtypeStruct((NPAD, EMB), F32)],
        interpret=interpret,
    )(x, dega, degb, m, w)


def _mid_body(a0_ref, a1_ref, dinv_ref, acc_ref, w_ref, accn_ref, u_ref):
    dinv = dinv_ref[...]
    tx1 = -dinv * (a0_ref[...] + a1_ref[...])
    accn_ref[...] = acc_ref[...] + jnp.dot(tx1, w_ref[...], preferred_element_type=F32)
    u_ref[...] = dinv * tx1


def _mid_call(a0, a1, dinv, acc, w, interpret=False):
    return pl.pallas_call(
        _mid_body,
        out_shape=[jax.ShapeDtypeStruct((NPAD, EMB), F32),
                   jax.ShapeDtypeStruct((NPAD, EMB), F32)],
        interpret=interpret,
    )(a0, a1, dinv, acc, w)


def _tail_common(a0, a1, dinv, acc, x0, w2, b, lw, lb, bg, bb, m):
    tx2 = -2.0 * dinv * (a0 + a1) - x0
    h = acc + jnp.dot(tx2, w2, preferred_element_type=F32) + b
    y = jnp.dot(h, lw, preferred_element_type=F32) + lb
    y = jnp.maximum(y, 0.0)
    cnt = jnp.maximum(jnp.sum(m), 1.0)
    mean = jnp.sum(y * m, axis=0, keepdims=True) / cnt
    var = jnp.sum(((y - mean) ** 2) * m, axis=0, keepdims=True) / cnt
    return bg * (y - mean) * lax.rsqrt(var + 1e-5) + bb


def _tail0_body(a0_ref, a1_ref, dinv_ref, acc_ref, x0_ref, w2_ref, b_ref,
                lw_ref, lb_ref, bg_ref, bb_ref, m_ref, wn_ref,
                x1_ref, u_ref, accn_ref):
    dinv = dinv_ref[...]
    xbn = _tail_common(a0_ref[...], a1_ref[...], dinv, acc_ref[...], x0_ref[...],
                       w2_ref[...], b_ref[...], lw_ref[...], lb_ref[...],
                       bg_ref[...], bb_ref[...], m_ref[...])
    x1_ref[...] = xbn
    u_ref[...] = dinv * xbn
    accn_ref[...] = jnp.dot(xbn, wn_ref[...], preferred_element_type=F32)


def _tail0_call(a0, a1, dinv, acc, x0, w2, b, lw, lb, bg, bb, m, wn, interpret=False):
    return pl.pallas_call(
        _tail0_body,
        out_shape=[jax.ShapeDtypeStruct((NPAD, EMB), F32),
                   jax.ShapeDtypeStruct((NPAD, EMB), F32),
                   jax.ShapeDtypeStruct((NPAD, EMB), F32)],
        interpret=interpret,
    )(a0, a1, dinv, acc, x0, w2, b, lw, lb, bg, bb, m, wn)


def _gather_graph_i32(oh, v):
    hi16 = lax.shift_right_arithmetic(v, 16)
    lo16 = v & jnp.int32(0xFFFF)
    hl = jnp.concatenate([hi16, lo16], axis=0).astype(F32)
    gn = jnp.dot(oh, hl.T, precision=HIP,
                 preferred_element_type=F32).astype(jnp.int32)
    return (gn[:, 0:1] << 16) | gn[:, 1:2]


def _colsum(vec, oh):
    return lax.dot_general(vec, oh, (((0,), (0,)), ((), ())), precision=HIP,
                           preferred_element_type=F32)


def _score_body(xbn_ref, pw_ref, m_ref, score_ref, key_ref):
    pw = pw_ref[...]
    wnorm = jnp.sqrt(jnp.sum(pw * pw))
    score = jnp.tanh(jnp.sum(xbn_ref[...] * pw, axis=1, keepdims=True) / wnorm)
    kb = lax.bitcast_convert_type(score, jnp.int32)
    key = jnp.where(kb < 0, kb ^ jnp.int32(0x7FFFFFFF), kb)
    score_ref[...] = score
    key_ref[...] = jnp.where(m_ref[...] > 0, key, jnp.int32(-2147483648))


def _score_call(xbn, pw, m, interpret=False):
    return pl.pallas_call(
        _score_body,
        out_shape=[jax.ShapeDtypeStruct((NPAD, 1), F32),
                   jax.ShapeDtypeStruct((NPAD, 1), jnp.int32)],
        interpret=interpret,
    )(xbn, pw, m)


def _thresh_body(key_ref, m_ref, batch_ref, t_ref, kk_ref):
    key = key_ref[...]
    oh = (batch_ref[...] == lax.broadcasted_iota(jnp.int32, (NPAD, G), 1)).astype(F32)
    size = _colsum(m_ref[...], oh)
    kk = jnp.floor((size + 1.0) * 0.5)

    def body(_, lh):
        lo, hi = lh
        mid = lo + (hi - lo) // 2
        midn = _gather_graph_i32(oh, mid)
        ind = jnp.where(key >= midn, 1.0, 0.0)
        cnt = _colsum(ind, oh)
        cond = cnt >= kk
        return jnp.where(cond, mid, lo), jnp.where(cond, hi, mid)

    lo0 = jnp.full((1, G), _KEY_N1, jnp.int32)
    hi0 = jnp.full((1, G), _KEY_P1 + 1, jnp.int32)
    t, _ = lax.fori_loop(0, 32, body, (lo0, hi0))
    t_ref[...] = t
    kk_ref[...] = kk


def _thresh_call(key, m, batch, interpret=False):
    return pl.pallas_call(
        _thresh_body,
        out_shape=[jax.ShapeDtypeStruct((1, G), jnp.int32),
                   jax.ShapeDtypeStruct((1, G), F32)],
        interpret=interpret,
    )(key, m, batch)


def _tiecnt_body(key_ref, batch_ref, t_ref, kk_ref, code_ref, pn_ref):
    key = key_ref[...]
    oh = (batch_ref[...] == lax.broadcasted_iota(jnp.int32, (NPAD, G), 1)).astype(F32)
    tn = _gather_graph_i32(oh, t_ref[...])
    gtb = (key > tn).astype(F32)
    tieb = (key == tn).astype(F32)
    cnt_gt = _colsum(gtb, oh)
    cnt_tie = _colsum(tieb, oh)
    need = kk_ref[...] - cnt_gt
    gg = lax.broadcasted_iota(jnp.int32, (G, G), 0)
    su = (gg < lax.broadcasted_iota(jnp.int32, (G, G), 1)).astype(F32)
    off = jnp.dot(cnt_tie, su, precision=HIP, preferred_element_type=F32)
    code_ref[...] = 2.0 * gtb + tieb
    pn_ref[...] = jnp.dot(oh, (off + need).T, precision=HIP,
                          preferred_element_type=F32)


def _tiecnt_call(key, batch, t, kk, interpret=False):
    return pl.pallas_call(
        _tiecnt_body,
        out_shape=[jax.ShapeDtypeStruct((NPAD, 1), F32),
                   jax.ShapeDtypeStruct((NPAD, 1), F32)],
        interpret=interpret,
    )(key, batch, t, kk)


def _keep_body(code_ref, pn_ref, keep_ref):
    code = code_ref[...]
    tie_i = jnp.where((code == 1.0) | (code == 3.0), 1.0, 0.0)

    def cs_body(i, cs):
        d = jnp.int32(1) << i
        return cs + jnp.where(_rows(NPAD, 1) < d, 0.0, pltpu.roll(cs, d, 0))

    cs = lax.fori_loop(0, 14, cs_body, tie_i)
    keep = (code >= 2.0) | ((tie_i > 0) & (cs - tie_i < pn_ref[...]))
    keep_ref[...] = keep.astype(F32)


def _keep_call(code, pn, interpret=False):
    return pl.pallas_call(
        _keep_body,
        out_shape=jax.ShapeDtypeStruct((NPAD, 1), F32),
        interpret=interpret,
    )(code, pn)


def _gap_body(xbn_ref, score_ref, keep_ref, batch_ref, xp_ref, gapcnt_ref):
    keepf = keep_ref[...]
    xp = xbn_ref[...] * score_ref[...] * keepf
    oh = (batch_ref[...] == lax.broadcasted_iota(jnp.int32, (NPAD, G), 1)).astype(F32)
    cnt_a = lax.dot_general(oh, keepf, (((0,), (0,)), ((), ())), precision=HIP,
                            preferred_element_type=F32)
    ssum = lax.dot_general(oh, xp, (((0,), (0,)), ((), ())), precision=HIP,
                           preferred_element_type=F32)
    gap = ssum / jnp.maximum(cnt_a, 1.0)
    xp_ref[...] = xp
    gapcnt_ref[...] = jnp.concatenate([gap, cnt_a], axis=1)


def _gap_call(xbn, score, keep, batch, interpret=False):
    return pl.pallas_call(
        _gap_body,
        out_shape=[jax.ShapeDtypeStruct((NPAD, EMB), F32),
                   jax.ShapeDtypeStruct((G, EMB + 1), F32)],
        interpret=interpret,
    )(xbn, score, keep, batch)


def _gmp_body(xp_ref, keep_ref, batch_ref, gapcnt_ref, repacc_ref, rep_ref):
    batch = batch_ref[...]
    xp = xp_ref[...]

    def scan_body(i, f):
        d = jnp.int32(1) << i
        bs = jnp.where(_rows(NPAD, 1) < d, jnp.int32(-1), pltpu.roll(batch, d, 0))
        fs = jnp.where(_rows(NPAD, EMB) < d, -1e30, pltpu.roll(f, d, 0))
        return jnp.where(bs == batch, jnp.maximum(f, fs), f)

    f = lax.fori_loop(0, 14, scan_body,
                      jnp.where(keep_ref[...] > 0, xp, -1e30))
    oh = (batch == lax.broadcasted_iota(jnp.int32, (NPAD, G), 1)).astype(F32)
    lastf = (batch != _shift_up(batch, 1, jnp.int32(-2))).astype(F32)
    gmp = lax.dot_general(oh * lastf, f, (((0,), (0,)), ((), ())), precision=HIP,
                          preferred_element_type=F32)
    gapcnt = gapcnt_ref[...]
    cnt_a = gapcnt[:, EMB:EMB + 1]
    gmp = jnp.where(cnt_a > 0.5, gmp, 0.0)
    rep = jnp.concatenate([gmp, gapcnt[:, :EMB]], axis=1)
    rep_ref[...] = repacc_ref[...] + rep


def _gmp_call(xp, keep, batch, gapcnt, repacc, interpret=False):
    return pl.pallas_call(
        _gmp_body,
        out_shape=jax.ShapeDtypeStruct((G, 2 * EMB), F32),
        interpret=interpret,
    )(xp, keep, batch, gapcnt, repacc)


def _pool_call(xbn, m, batch, pw, repacc, interpret=False):
    score, key = _score_call(xbn, pw, m, interpret=interpret)
    t, kk = _thresh_call(key, m, batch, interpret=interpret)
    code, pn = _tiecnt_call(key, batch, t, kk, interpret=interpret)
    keep = _keep_call(code, pn, interpret=interpret)
    xp, gapcnt = _gap_call(xbn, score, keep, batch, interpret=interpret)
    rep = _gmp_call(xp, keep, batch, gapcnt, repacc, interpret=interpret)
    return xp, keep, rep


def _tailbn_body(a0_ref, a1_ref, dinv_ref, acc_ref, x0_ref, w2_ref, b_ref,
                 lw_ref, lb_ref, bg_ref, bb_ref, m_ref, xbn_ref):
    xbn_ref[...] = _tail_common(a0_ref[...], a1_ref[...], dinv_ref[...],
                                acc_ref[...], x0_ref[...], w2_ref[...],
                                b_ref[...], lw_ref[...], lb_ref[...],
                                bg_ref[...], bb_ref[...], m_ref[...])


def _tailbn_call(a0, a1, dinv, acc, x0, w2, b, lw, lb, bg, bb, m, interpret=False):
    return pl.pallas_call(
        _tailbn_body,
        out_shape=jax.ShapeDtypeStruct((NPAD, EMB), F32),
        interpret=interpret,
    )(a0, a1, dinv, acc, x0, w2, b, lw, lb, bg, bb, m)


def _zero_vmem_rows(buf, nrows, width):
    z = jnp.zeros((16,), F32)

    def row(i, _):
        for j in range(width // 16):
            buf[i, pl.ds(j * 16, 16)] = z
        return 0

    lax.fori_loop(0, nrows, row, 0)


@functools.lru_cache(maxsize=None)
def _sc_kernels():
    mesh = plsc.VectorSubcoreMesh(core_axis_name="c", subcore_axis_name="s")

    @functools.partial(
        pl.kernel, mesh=mesh,
        out_type=[jax.ShapeDtypeStruct((NPAD, EMB), F32),
                  jax.ShapeDtypeStruct((NPAD, EMB), F32)],
        scratch_types=[pltpu.VMEM((CHUNK,), jnp.int32),
                       pltpu.VMEM((CHUNK,), jnp.int32),
                       pltpu.VMEM((CHUNK,), jnp.int32),
                       pltpu.VMEM((CHUNK,), jnp.int32),
                       pltpu.VMEM((CHUNK, EMB), F32),
                       pltpu.VMEM((CHUNK, EMB), F32),
                       pltpu.VMEM_SHARED((NPAD, EMB), F32),
                       pltpu.SemaphoreType.DMA,
                       pltpu.SemaphoreType.DMA,
                       pltpu.SemaphoreType.DMA],
    )
    def _lap_kernel(u_hbm, src_hbm, dst_hbm, o0_hbm, o1_hbm,
                    si0, di0, si1, di1, rows0, rows1, acc, semG, semS0, semS1):
        cid = lax.axis_index("c")
        sid = lax.axis_index("s")
        _zero_vmem_rows(rows0, CHUNK, EMB)
        for j in range(STRIPE // CHUNK):
            pltpu.sync_copy(rows0, acc.at[pl.ds(sid * STRIPE + j * CHUNK, CHUNK)])
        plsc.subcore_barrier()

        base = cid * EPC + sid * EPW

        def chunk(i, si, di, rows, semS, wait_prev):
            pltpu.sync_copy(src_hbm.at[pl.ds(base + i * CHUNK, CHUNK)], si)
            pltpu.sync_copy(dst_hbm.at[pl.ds(base + i * CHUNK, CHUNK)], di)
            pltpu.async_copy(u_hbm.at[si], rows, semG)
            if wait_prev:
                pltpu.make_async_copy(rows, acc.at[di], semS).wait()
            pltpu.make_async_copy(u_hbm.at[si], rows, semG).wait()
            pltpu.async_copy(rows, acc.at[di], semS, add=True)

        chunk(0, si0, di0, rows0, semS0, False)
        chunk(1, si1, di1, rows1, semS1, False)

        def step(j, _):
            chunk(2 * j, si0, di0, rows0, semS0, True)
            chunk(2 * j + 1, si1, di1, rows1, semS1, True)
            return 0

        lax.fori_loop(1, NCH // 2, step, 0)
        pltpu.make_async_copy(rows0, acc.at[di0], semS0).wait()
        pltpu.make_async_copy(rows1, acc.at[di1], semS1).wait()
        plsc.subcore_barrier()

        @pl.when(cid == 0)
        def _():
            pltpu.sync_copy(acc.at[pl.ds(sid * STRIPE, STRIPE)],
                            o0_hbm.at[pl.ds(sid * STRIPE, STRIPE)])

        @pl.when(cid == 1)
        def _():
            pltpu.sync_copy(acc.at[pl.ds(sid * STRIPE, STRIPE)],
                            o1_hbm.at[pl.ds(sid * STRIPE, STRIPE)])

    return (_lap_kernel,)


def _lap_call(u, edges):
    src1, dst1 = edges
    return _sc_kernels()[0](u, src1, dst1)


def _mdeg_call(m128, edges):
    src1, dst1 = edges
    return _sc_kernels()[0](m128, dst1, src1)


def _forward(x, edge_index, batch_index, cheb_W, cheb_b, lin_W, lin_b,
             bn_g, bn_b, pool_w, lap_fn, mdeg_fn, interpret=False):
    xp = jnp.zeros((NPAD, EMB), F32).at[:NREAL].set(x)
    batch = jnp.full((NPAD, 1), G - 1, jnp.int32).at[:NREAL, 0].set(batch_index)
    m = jnp.zeros((NPAD, 1), F32).at[:NREAL].set(1.0)
    m128 = jnp.zeros((NPAD, EMB), F32).at[:NREAL].set(1.0)
    pad = jnp.full((EPAD - E,), NREAL, jnp.int32)
    src1 = jnp.concatenate([edge_index[0], pad])
    dst1 = jnp.concatenate([edge_index[1], pad])
    edges = (src1, dst1)
    b2 = lambda v: v.reshape(1, EMB)

    def conv_to_tail(xin, dega, degb, m, w3):
        dinv, u, acc = _pre_call(xin, dega, degb, m, w3[0], interpret=interpret)
        a0, a1 = lap_fn(u, edges)
        acc, u = _mid_call(a0, a1, dinv, acc, w3[1], interpret=interpret)
        a0, a1 = lap_fn(u, edges)
        return dinv, acc, a0, a1

    rep = jnp.zeros((G, 2 * EMB), F32)
    dega, degb = mdeg_fn(m128, edges)
    dinv, acc, a0, a1 = conv_to_tail(xp, dega, degb, m, cheb_W[0])
    xp, u, acc = _tail0_call(a0, a1, dinv, acc, xp, cheb_W[0][2], b2(cheb_b[0]),
                             lin_W[0], b2(lin_b[0]), b2(bn_g[0]), b2(bn_b[0]),
                             m, cheb_W[1][0], interpret=interpret)
    a0, a1 = lap_fn(u, edges)
    acc, u = _mid_call(a0, a1, dinv, acc, cheb_W[1][1], interpret=interpret)
    a0, a1 = lap_fn(u, edges)
    xbn = _tailbn_call(a0, a1, dinv, acc, xp, cheb_W[1][2], b2(cheb_b[1]),
                       lin_W[1], b2(lin_b[1]), b2(bn_g[1]), b2(bn_b[1]), m,
                       interpret=interpret)
    xp, m, rep = _pool_call(xbn, m, batch, b2(pool_w[0]), rep,
                            interpret=interpret)
    m128 = jnp.broadcast_to(m, (NPAD, EMB))
    for i in (1, 2):
        dega, degb = mdeg_fn(m128, edges)
        dinv, acc, a0, a1 = conv_to_tail(xp, dega, degb, m, cheb_W[i + 1])
        xbn = _tailbn_call(a0, a1, dinv, acc, xp, cheb_W[i + 1][2],
                           b2(cheb_b[i + 1]), lin_W[i + 1], b2(lin_b[i + 1]),
                           b2(bn_g[i + 1]), b2(bn_b[i + 1]), m,
                           interpret=interpret)
        xp, m, rep = _pool_call(xbn, m, batch, b2(pool_w[i]), rep,
                                interpret=interpret)
        m128 = jnp.broadcast_to(m, (NPAD, EMB))
    return rep


def kernel(x, edge_attr, edge_index, batch_index, cheb_W, cheb_b, lin_W, lin_b,
           bn_g, bn_b, pool_w):
    return _forward(x, edge_index, batch_index, cheb_W, cheb_b, lin_W, lin_b,
                    bn_g, bn_b, pool_w, _lap_call, _mdeg_call)


# speedup vs baseline: 1.4487x; 1.0951x over previous
import functools

import jax
import jax.numpy as jnp
from jax import lax
from jax.experimental import pallas as pl
from jax.experimental.pallas import tpu as pltpu
from jax.experimental.pallas import tpu_sc as plsc

NREAL = 10000
NPAD = 10240
E = 320000
EMB = 128
G = 128
HIP = lax.Precision.HIGHEST
F32 = jnp.float32

NC, NS = 2, 16
CHUNK = 80
EPC = E // NC
EPW = EPC // NS
NCH = EPW // CHUNK
STRIPE = NPAD // NS

_KEY_N1 = -1065353217
_KEY_P1 = 1065353216


def _rows(n, w=1):
    return lax.broadcasted_iota(jnp.int32, (n, w), 0)


def _shift_down(a, d, fill):
    r = pltpu.roll(a, d, 0)
    return jnp.where(_rows(a.shape[0], a.shape[1]) < d, fill, r)


def _shift_up(a, d, fill):
    r = pltpu.roll(a, a.shape[0] - d, 0)
    return jnp.where(_rows(a.shape[0], a.shape[1]) >= a.shape[0] - d, fill, r)


def _pre_body(x_ref, dega_ref, degb_ref, m_ref, w_ref, dinv_ref, u_ref, acc_ref):
    x = x_ref[...]
    deg = (dega_ref[:, 0:1] + degb_ref[:, 0:1]) * m_ref[...]
    dinv = jnp.where(deg > 0, lax.rsqrt(jnp.maximum(deg, 1e-30)), 0.0)
    dinv_ref[...] = dinv
    u_ref[...] = x * dinv
    acc_ref[...] = jnp.dot(x, w_ref[...], preferred_element_type=F32)


def _pre_call(x, dega, degb, m, w, interpret=False):
    return pl.pallas_call(
        _pre_body,
        out_shape=[jax.ShapeDtypeStruct((NPAD, 1), F32),
                   jax.ShapeDtypeStruct((NPAD, EMB), F32),
                   jax.ShapeDtypeStruct((NPAD, EMB), F32)],
        interpret=interpret,
    )(x, dega, degb, m, w)


def _mid_body(a0_ref, a1_ref, dinv_ref, acc_ref, w_ref, accn_ref, u_ref):
    dinv = dinv_ref[...]
    tx1 = -dinv * (a0_ref[...] + a1_ref[...])
    accn_ref[...] = acc_ref[...] + jnp.dot(tx1, w_ref[...], preferred_element_type=F32)
    u_ref[...] = dinv * tx1


def _mid_call(a0, a1, dinv, acc, w, interpret=False):
    return pl.pallas_call(
        _mid_body,
        out_shape=[jax.ShapeDtypeStruct((NPAD, EMB), F32),
                   jax.ShapeDtypeStruct((NPAD, EMB), F32)],
        interpret=interpret,
    )(a0, a1, dinv, acc, w)


def _tail_common(a0, a1, dinv, acc, x0, w2, b, lw, lb, bg, bb, m):
    tx2 = -2.0 * dinv * (a0 + a1) - x0
    h = acc + jnp.dot(tx2, w2, preferred_element_type=F32) + b
    y = jnp.dot(h, lw, preferred_element_type=F32) + lb
    y = jnp.maximum(y, 0.0)
    cnt = jnp.maximum(jnp.sum(m), 1.0)
    mean = jnp.sum(y * m, axis=0, keepdims=True) / cnt
    var = jnp.sum(((y - mean) ** 2) * m, axis=0, keepdims=True) / cnt
    return bg * (y - mean) * lax.rsqrt(var + 1e-5) + bb


def _tail0_body(a0_ref, a1_ref, dinv_ref, acc_ref, x0_ref, w2_ref, b_ref,
                lw_ref, lb_ref, bg_ref, bb_ref, m_ref, wn_ref,
                x1_ref, u_ref, accn_ref):
    dinv = dinv_ref[...]
    xbn = _tail_common(a0_ref[...], a1_ref[...], dinv, acc_ref[...], x0_ref[...],
                       w2_ref[...], b_ref[...], lw_ref[...], lb_ref[...],
                       bg_ref[...], bb_ref[...], m_ref[...])
    x1_ref[...] = xbn
    u_ref[...] = dinv * xbn
    accn_ref[...] = jnp.dot(xbn, wn_ref[...], preferred_element_type=F32)


def _tail0_call(a0, a1, dinv, acc, x0, w2, b, lw, lb, bg, bb, m, wn, interpret=False):
    return pl.pallas_call(
        _tail0_body,
        out_shape=[jax.ShapeDtypeStruct((NPAD, EMB), F32),
                   jax.ShapeDtypeStruct((NPAD, EMB), F32),
                   jax.ShapeDtypeStruct((NPAD, EMB), F32)],
        interpret=interpret,
    )(a0, a1, dinv, acc, x0, w2, b, lw, lb, bg, bb, m, wn)


def _gather_graph_i32(oh, v):
    hi16 = lax.shift_right_arithmetic(v, 16)
    lo16 = v & jnp.int32(0xFFFF)
    hl = jnp.concatenate([hi16, lo16], axis=0).astype(F32)
    gn = jnp.dot(oh, hl.T, precision=HIP,
                 preferred_element_type=F32).astype(jnp.int32)
    return (gn[:, 0:1] << 16) | gn[:, 1:2]


def _colsum(vec, oh):
    return lax.dot_general(vec, oh, (((0,), (0,)), ((), ())), precision=HIP,
                           preferred_element_type=F32)


def _score_body(xbn_ref, pw_ref, m_ref, score_ref, key_ref):
    pw = pw_ref[...]
    wnorm = jnp.sqrt(jnp.sum(pw * pw))
    score = jnp.tanh(jnp.sum(xbn_ref[...] * pw, axis=1, keepdims=True) / wnorm)
    kb = lax.bitcast_convert_type(score, jnp.int32)
    key = jnp.where(kb < 0, kb ^ jnp.int32(0x7FFFFFFF), kb)
    score_ref[...] = score
    key_ref[...] = jnp.where(m_ref[...] > 0, key, jnp.int32(-2147483648))


def _score_call(xbn, pw, m, interpret=False):
    return pl.pallas_call(
        _score_body,
        out_shape=[jax.ShapeDtypeStruct((NPAD, 1), F32),
                   jax.ShapeDtypeStruct((NPAD, 1), jnp.int32)],
        interpret=interpret,
    )(xbn, pw, m)


def _thresh_body(key_ref, m_ref, batch_ref, t_ref, kk_ref):
    key = key_ref[...]
    oh = (batch_ref[...] == lax.broadcasted_iota(jnp.int32, (NPAD, G), 1)).astype(F32)
    size = _colsum(m_ref[...], oh)
    kk = jnp.floor((size + 1.0) * 0.5)

    def body(_, lh):
        lo, hi = lh
        mid = lo + (hi - lo) // 2
        midn = _gather_graph_i32(oh, mid)
        ind = jnp.where(key >= midn, 1.0, 0.0)
        cnt = _colsum(ind, oh)
        cond = cnt >= kk
        return jnp.where(cond, mid, lo), jnp.where(cond, hi, mid)

    lo0 = jnp.full((1, G), _KEY_N1, jnp.int32)
    hi0 = jnp.full((1, G), _KEY_P1 + 1, jnp.int32)
    t, _ = lax.fori_loop(0, 32, body, (lo0, hi0))
    t_ref[...] = t
    kk_ref[...] = kk


def _thresh_call(key, m, batch, interpret=False):
    return pl.pallas_call(
        _thresh_body,
        out_shape=[jax.ShapeDtypeStruct((1, G), jnp.int32),
                   jax.ShapeDtypeStruct((1, G), F32)],
        interpret=interpret,
    )(key, m, batch)


def _tiecnt_body(key_ref, batch_ref, t_ref, kk_ref, code_ref, pn_ref):
    key = key_ref[...]
    oh = (batch_ref[...] == lax.broadcasted_iota(jnp.int32, (NPAD, G), 1)).astype(F32)
    tn = _gather_graph_i32(oh, t_ref[...])
    gtb = (key > tn).astype(F32)
    tieb = (key == tn).astype(F32)
    cnt_gt = _colsum(gtb, oh)
    cnt_tie = _colsum(tieb, oh)
    need = kk_ref[...] - cnt_gt
    gg = lax.broadcasted_iota(jnp.int32, (G, G), 0)
    su = (gg < lax.broadcasted_iota(jnp.int32, (G, G), 1)).astype(F32)
    off = jnp.dot(cnt_tie, su, precision=HIP, preferred_element_type=F32)
    code_ref[...] = 2.0 * gtb + tieb
    pn_ref[...] = jnp.dot(oh, (off + need).T, precision=HIP,
                          preferred_element_type=F32)


def _tiecnt_call(key, batch, t, kk, interpret=False):
    return pl.pallas_call(
        _tiecnt_body,
        out_shape=[jax.ShapeDtypeStruct((NPAD, 1), F32),
                   jax.ShapeDtypeStruct((NPAD, 1), F32)],
        interpret=interpret,
    )(key, batch, t, kk)


def _keep_body(code_ref, pn_ref, keep_ref):
    code = code_ref[...]
    tie_i = jnp.where((code == 1.0) | (code == 3.0), 1.0, 0.0)

    def cs_body(i, cs):
        d = jnp.int32(1) << i
        return cs + jnp.where(_rows(NPAD, 1) < d, 0.0, pltpu.roll(cs, d, 0))

    cs = lax.fori_loop(0, 14, cs_body, tie_i)
    keep = (code >= 2.0) | ((tie_i > 0) & (cs - tie_i < pn_ref[...]))
    keep_ref[...] = keep.astype(F32)


def _keep_call(code, pn, interpret=False):
    return pl.pallas_call(
        _keep_body,
        out_shape=jax.ShapeDtypeStruct((NPAD, 1), F32),
        interpret=interpret,
    )(code, pn)


def _gap_body(xbn_ref, score_ref, keep_ref, batch_ref, xp_ref, gapcnt_ref):
    keepf = keep_ref[...]
    xp = xbn_ref[...] * score_ref[...] * keepf
    oh = (batch_ref[...] == lax.broadcasted_iota(jnp.int32, (NPAD, G), 1)).astype(F32)
    cnt_a = lax.dot_general(oh, keepf, (((0,), (0,)), ((), ())), precision=HIP,
                            preferred_element_type=F32)
    ssum = lax.dot_general(oh, xp, (((0,), (0,)), ((), ())), precision=HIP,
                           preferred_element_type=F32)
    gap = ssum / jnp.maximum(cnt_a, 1.0)
    xp_ref[...] = xp
    gapcnt_ref[...] = jnp.concatenate([gap, cnt_a], axis=1)


def _gap_call(xbn, score, keep, batch, interpret=False):
    return pl.pallas_call(
        _gap_body,
        out_shape=[jax.ShapeDtypeStruct((NPAD, EMB), F32),
                   jax.ShapeDtypeStruct((G, EMB + 1), F32)],
        interpret=interpret,
    )(xbn, score, keep, batch)


def _gmp_body(xp_ref, keep_ref, batch_ref, gapcnt_ref, repacc_ref, rep_ref):
    batch = batch_ref[...]
    xp = xp_ref[...]

    def scan_body(i, f):
        d = jnp.int32(1) << i
        bs = jnp.where(_rows(NPAD, 1) < d, jnp.int32(-1), pltpu.roll(batch, d, 0))
        fs = jnp.where(_rows(NPAD, EMB) < d, -1e30, pltpu.roll(f, d, 0))
        return jnp.where(bs == batch, jnp.maximum(f, fs), f)

    f = lax.fori_loop(0, 14, scan_body,
                      jnp.where(keep_ref[...] > 0, xp, -1e30))
    oh = (batch == lax.broadcasted_iota(jnp.int32, (NPAD, G), 1)).astype(F32)
    lastf = (batch != _shift_up(batch, 1, jnp.int32(-2))).astype(F32)
    gmp = lax.dot_general(oh * lastf, f, (((0,), (0,)), ((), ())), precision=HIP,
                          preferred_element_type=F32)
    gapcnt = gapcnt_ref[...]
    cnt_a = gapcnt[:, EMB:EMB + 1]
    gmp = jnp.where(cnt_a > 0.5, gmp, 0.0)
    rep = jnp.concatenate([gmp, gapcnt[:, :EMB]], axis=1)
    rep_ref[...] = repacc_ref[...] + rep


def _gmp_call(xp, keep, batch, gapcnt, repacc, interpret=False):
    return pl.pallas_call(
        _gmp_body,
        out_shape=jax.ShapeDtypeStruct((G, 2 * EMB), F32),
        interpret=interpret,
    )(xp, keep, batch, gapcnt, repacc)


def _pool_call(xbn, m, batch, pw, repacc, interpret=False):
    score, key = _score_call(xbn, pw, m, interpret=interpret)
    t, kk = _thresh_call(key, m, batch, interpret=interpret)
    code, pn = _tiecnt_call(key, batch, t, kk, interpret=interpret)
    keep = _keep_call(code, pn, interpret=interpret)
    xp, gapcnt = _gap_call(xbn, score, keep, batch, interpret=interpret)
    rep = _gmp_call(xp, keep, batch, gapcnt, repacc, interpret=interpret)
    return xp, keep, rep


def _tailbn_body(a0_ref, a1_ref, dinv_ref, acc_ref, x0_ref, w2_ref, b_ref,
                 lw_ref, lb_ref, bg_ref, bb_ref, m_ref, xbn_ref):
    xbn_ref[...] = _tail_common(a0_ref[...], a1_ref[...], dinv_ref[...],
                                acc_ref[...], x0_ref[...], w2_ref[...],
                                b_ref[...], lw_ref[...], lb_ref[...],
                                bg_ref[...], bb_ref[...], m_ref[...])


def _tailbn_call(a0, a1, dinv, acc, x0, w2, b, lw, lb, bg, bb, m, interpret=False):
    return pl.pallas_call(
        _tailbn_body,
        out_shape=jax.ShapeDtypeStruct((NPAD, EMB), F32),
        interpret=interpret,
    )(a0, a1, dinv, acc, x0, w2, b, lw, lb, bg, bb, m)


def _zero_vmem_rows(buf, nrows, width):
    z = jnp.zeros((16,), F32)

    def row(i, _):
        for j in range(width // 16):
            buf[i, pl.ds(j * 16, 16)] = z
        return 0

    lax.fori_loop(0, nrows, row, 0)


@functools.lru_cache(maxsize=None)
def _sc_kernels():
    mesh = plsc.VectorSubcoreMesh(core_axis_name="c", subcore_axis_name="s")

    @functools.partial(
        pl.kernel, mesh=mesh,
        out_type=[jax.ShapeDtypeStruct((NPAD, EMB), F32),
                  jax.ShapeDtypeStruct((NPAD, EMB), F32)],
        scratch_types=[pltpu.VMEM((CHUNK,), jnp.int32),
                       pltpu.VMEM((CHUNK,), jnp.int32),
                       pltpu.VMEM((CHUNK, EMB), F32),
                       pltpu.VMEM_SHARED((NPAD, EMB), F32),
                       pltpu.SemaphoreType.DMA],
    )
    def _lap_kernel(u_hbm, src_hbm, dst_hbm, o0_hbm, o1_hbm, si, di, rows, acc, sem):
        cid = lax.axis_index("c")
        sid = lax.axis_index("s")
        _zero_vmem_rows(rows, CHUNK, EMB)
        for j in range(STRIPE // CHUNK):
            pltpu.sync_copy(rows, acc.at[pl.ds(sid * STRIPE + j * CHUNK, CHUNK)])
        plsc.subcore_barrier()

        base = cid * EPC + sid * EPW

        def step(i, _):
            pltpu.sync_copy(src_hbm.at[pl.ds(base + i * CHUNK, CHUNK)], si)
            pltpu.sync_copy(dst_hbm.at[pl.ds(base + i * CHUNK, CHUNK)], di)
            pltpu.async_copy(u_hbm.at[si], rows, sem).wait()
            pltpu.sync_copy(rows, acc.at[di], add=True)
            return 0

        lax.fori_loop(0, NCH, step, 0)
        plsc.subcore_barrier()

        @pl.when(cid == 0)
        def _():
            pltpu.sync_copy(acc.at[pl.ds(sid * STRIPE, STRIPE)],
                            o0_hbm.at[pl.ds(sid * STRIPE, STRIPE)])

        @pl.when(cid == 1)
        def _():
            pltpu.sync_copy(acc.at[pl.ds(sid * STRIPE, STRIPE)],
                            o1_hbm.at[pl.ds(sid * STRIPE, STRIPE)])

    return (_lap_kernel,)


def _lap_call(u, edges):
    src1, dst1 = edges
    return _sc_kernels()[0](u, src1, dst1)


def _mdeg_call(m128, edges):
    src1, dst1 = edges
    return _sc_kernels()[0](m128, dst1, src1)


def _forward(x, edge_index, batch_index, cheb_W, cheb_b, lin_W, lin_b,
             bn_g, bn_b, pool_w, lap_fn, mdeg_fn, interpret=False):
    xp = jnp.zeros((NPAD, EMB), F32).at[:NREAL].set(x)
    batch = jnp.full((NPAD, 1), G - 1, jnp.int32).at[:NREAL, 0].set(batch_index)
    m = jnp.zeros((NPAD, 1), F32).at[:NREAL].set(1.0)
    m128 = jnp.zeros((NPAD, EMB), F32).at[:NREAL].set(1.0)
    edges = (edge_index[0], edge_index[1])
    b2 = lambda v: v.reshape(1, EMB)

    def conv_to_tail(xin, dega, degb, m, w3):
        dinv, u, acc = _pre_call(xin, dega, degb, m, w3[0], interpret=interpret)
        a0, a1 = lap_fn(u, edges)
        acc, u = _mid_call(a0, a1, dinv, acc, w3[1], interpret=interpret)
        a0, a1 = lap_fn(u, edges)
        return dinv, acc, a0, a1

    rep = jnp.zeros((G, 2 * EMB), F32)
    dega, degb = mdeg_fn(m128, edges)
    dinv, acc, a0, a1 = conv_to_tail(xp, dega, degb, m, cheb_W[0])
    xp, u, acc = _tail0_call(a0, a1, dinv, acc, xp, cheb_W[0][2], b2(cheb_b[0]),
                             lin_W[0], b2(lin_b[0]), b2(bn_g[0]), b2(bn_b[0]),
                             m, cheb_W[1][0], interpret=interpret)
    a0, a1 = lap_fn(u, edges)
    acc, u = _mid_call(a0, a1, dinv, acc, cheb_W[1][1], interpret=interpret)
    a0, a1 = lap_fn(u, edges)
    xbn = _tailbn_call(a0, a1, dinv, acc, xp, cheb_W[1][2], b2(cheb_b[1]),
                       lin_W[1], b2(lin_b[1]), b2(bn_g[1]), b2(bn_b[1]), m,
                       interpret=interpret)
    xp, m, rep = _pool_call(xbn, m, batch, b2(pool_w[0]), rep,
                            interpret=interpret)
    m128 = jnp.broadcast_to(m, (NPAD, EMB))
    for i in (1, 2):
        dega, degb = mdeg_fn(m128, edges)
        dinv, acc, a0, a1 = conv_to_tail(xp, dega, degb, m, cheb_W[i + 1])
        xbn = _tailbn_call(a0, a1, dinv, acc, xp, cheb_W[i + 1][2],
                           b2(cheb_b[i + 1]), lin_W[i + 1], b2(lin_b[i + 1]),
                           b2(bn_g[i + 1]), b2(bn_b[i + 1]), m,
                           interpret=interpret)
        xp, m, rep = _pool_call(xbn, m, batch, b2(pool_w[i]), rep,
                                interpret=interpret)
        m128 = jnp.broadcast_to(m, (NPAD, EMB))
    return rep


def kernel(x, edge_attr, edge_index, batch_index, cheb_W, cheb_b, lin_W, lin_b,
           bn_g, bn_b, pool_w):
    return _forward(x, edge_index, batch_index, cheb_W, cheb_b, lin_W, lin_b,
                    bn_g, bn_b, pool_w, _lap_call, _mdeg_call)
